# Initial kernel scaffold; baseline (speedup 1.0000x reference)
import jax
import jax.numpy as jnp
from jax.experimental import pallas as pl


def kernel(x, weight):
    raise NotImplementedError("write your pallas kernel here")


# speedup vs baseline: 5.0767x; 5.0767x over previous
import jax
import jax.numpy as jnp
from jax.experimental import pallas as pl
from jax.experimental.pallas import tpu as pltpu

_H = 512
_D = 128
_BASE = 65024
_WIN = 1024
_BI = 8


def _band_body(win_ref, x_ref, out_ref):
    i0 = pl.program_id(0) * _BI
    xb = x_ref[...]
    for ii in range(_BI):
        out_ref[ii] = win_ref[pl.ds(511 - (i0 + ii), _H), :] + xb


def kernel(x, weight):
    band = jax.lax.slice(weight, (_BASE, 0), (_BASE + _WIN, _D))
    xcol = x.reshape(_H, 1)
    out3 = pl.pallas_call(
        _band_body,
        grid=(_H // _BI,),
        in_specs=[
            pl.BlockSpec((_WIN, _D), lambda i: (0, 0)),
            pl.BlockSpec((_H, 1), lambda i: (0, 0)),
        ],
        out_specs=pl.BlockSpec((_BI, _H, _D), lambda i: (i, 0, 0)),
        out_shape=jax.ShapeDtypeStruct((_H, _H, _D), jnp.float32),
    )(band, xcol)
    return out3.reshape(1, _H, 1, _H, _D)
